# baseline (speedup 1.0000x reference)
import functools

import jax
import jax.numpy as jnp
import numpy as np
from jax import lax
from jax.experimental import pallas as pl
from jax.experimental.pallas import tpu as pltpu
from jax.experimental.pallas import tpu_sc as plsc

N_NODES_C = 100000
B_C = 50000
K_C = 10
D_C = 128
W_C = D_C // 2

CHUNK_B = 64
CHUNK_I = CHUNK_B * K_C
GATHER_SLICE = 128
N_GATHER = CHUNK_I // GATHER_SLICE
LANES = 16
H_VECS = W_C // LANES


def _col_interleave_perm():
    perm = np.empty(D_C, dtype=np.int32)
    for g in range(D_C // 32):
        for i in range(16):
            perm[32 * g + 2 * i] = 32 * g + i
            perm[32 * g + 2 * i + 1] = 32 * g + 16 + i
    return perm


_PERM = _col_interleave_perm()


def _make_sc_kernel():
    info = plsc.get_sparse_core_info()
    nc, ns = info.num_cores, info.num_subcores
    nw = nc * ns
    n_chunks = -(-B_C // CHUNK_B)
    per_w = -(-n_chunks // nw)
    last_base = B_C - CHUNK_B

    mesh = plsc.VectorSubcoreMesh(core_axis_name="c", subcore_axis_name="s")

    @functools.partial(
        pl.kernel,
        mesh=mesh,
        compiler_params=pltpu.CompilerParams(use_tc_tiling_on_sc=False),
        out_type=jax.ShapeDtypeStruct((B_C, D_C), jnp.float32),
        scratch_types=[
            pltpu.VMEM((CHUNK_I,), jnp.int32),
            pltpu.VMEM((CHUNK_I,), jnp.int32),
            pltpu.VMEM((CHUNK_I, W_C), jnp.int32),
            pltpu.VMEM((CHUNK_I, W_C), jnp.int32),
            pltpu.VMEM((CHUNK_B, D_C), jnp.float32),
            pltpu.VMEM((CHUNK_B, D_C), jnp.float32),
            pltpu.SemaphoreType.DMA,
            pltpu.SemaphoreType.DMA,
            pltpu.SemaphoreType.DMA,
        ],
    )
    def sc_kernel(table_hbm, neigh_hbm, out_hbm, idx_a, idx_b,
                  rows_a, rows_b, out_a, out_b, isem, gsem, osem):
        wid = lax.axis_index("s") * nc + lax.axis_index("c")
        inv_k = jnp.float32(1.0 / K_C)
        himask = jnp.int32(-65536)

        def lo_f32(v):
            return lax.bitcast_convert_type(lax.shift_left(v, 16),
                                            jnp.float32)

        def hi_f32(v):
            return lax.bitcast_convert_type(lax.bitwise_and(v, himask),
                                            jnp.float32)

        def chunk_base(j):
            return jnp.minimum((wid * per_w + j) * CHUNK_B, last_base)

        def issue_idx(j, idx_ref):
            base = chunk_base(j)
            pltpu.async_copy(
                neigh_hbm.at[pl.ds(base * K_C, CHUNK_I)], idx_ref, isem)

        def drain_idx():
            pltpu.make_async_copy(
                neigh_hbm.at[pl.ds(0, CHUNK_I)], idx_a, isem).wait()

        def issue_gathers(idx_ref, rows_ref):
            for g in range(N_GATHER):
                sl = pl.ds(g * GATHER_SLICE, GATHER_SLICE)
                pltpu.async_copy(
                    table_hbm.at[idx_ref.at[sl]], rows_ref.at[sl], gsem)

        def drain_gathers():
            pltpu.make_async_copy(
                table_hbm.at[pl.ds(0, CHUNK_I)], rows_a, gsem).wait()

        def issue_out(j, out_ref):
            base = chunk_base(j)
            pltpu.async_copy(
                out_ref, out_hbm.at[pl.ds(base, CHUNK_B)], osem)

        def drain_out():
            pltpu.make_async_copy(
                out_a, out_hbm.at[pl.ds(0, CHUNK_B)], osem).wait()

        def reduce_chunk(rv, ov):
            def tree_ops(l, b, h):
                t = {}
                lo_sl = pl.ds(32 * h, LANES)
                hi_sl = pl.ds(32 * h + LANES, LANES)

                def mk(name, f):
                    return lambda: t.__setitem__(name, f())

                ops = []
                for half, cvt in (("e", lo_f32), ("o", hi_f32)):
                    ops += [
                        mk(half + "a0", lambda c=cvt: c(l[0]) + c(l[1])),
                        mk(half + "a1", lambda c=cvt: c(l[2]) + c(l[3])),
                        mk(half + "a2", lambda c=cvt: c(l[4]) + c(l[5])),
                        mk(half + "a3", lambda c=cvt: c(l[6]) + c(l[7])),
                        mk(half + "a4", lambda c=cvt: c(l[8]) + c(l[9])),
                        mk(half + "b0",
                           lambda p=half: t[p + "a0"] + t[p + "a1"]),
                        mk(half + "b1",
                           lambda p=half: t[p + "a2"] + t[p + "a3"]),
                        mk(half + "c0",
                           lambda p=half: t[p + "b0"] + t[p + "b1"]),
                        mk(half + "c1",
                           lambda p=half: t[p + "c0"] + t[p + "a4"]),
                    ]
                ops.append(lambda: ov.__setitem__(
                    (b, lo_sl), t["ec1"] * inv_k))
                ops.append(lambda: ov.__setitem__(
                    (b, hi_sl), t["oc1"] * inv_k))
                return ops

            def center_body(i, carry2):
                b = i
                r0 = b * K_C
                pending = []
                for h in range(H_VECS):
                    hsl = pl.ds(h * LANES, LANES)
                    loads = []
                    for k in range(K_C):
                        loads.append(rv[r0 + k, hsl])
                        for _ in range(2):
                            if pending:
                                pending.pop(0)()
                    pending = tree_ops(loads, b, h)
                for op in pending:
                    op()
                return carry2

            lax.fori_loop(0, CHUNK_B, center_body, 0)

        issue_idx(0, idx_a)
        drain_idx()
        issue_idx(1, idx_b)
        issue_gathers(idx_a, rows_a)

        def chunk_body(j, carry):
            r = lax.rem(j, 2)
            nr = 1 - r

            drain_gathers()

            @pl.when(j < per_w - 2)
            def _():
                @pl.when(r == 0)
                def _():
                    issue_idx(j + 2, idx_a)

                @pl.when(r == 1)
                def _():
                    issue_idx(j + 2, idx_b)

            @pl.when(j < per_w - 1)
            def _():
                drain_idx()

                @pl.when(nr == 0)
                def _():
                    issue_gathers(idx_a, rows_a)

                @pl.when(nr == 1)
                def _():
                    issue_gathers(idx_b, rows_b)

            @pl.when(j >= 2)
            def _():
                drain_out()

            @pl.when(r == 0)
            def _():
                reduce_chunk(rows_a, out_a)
                issue_out(j, out_a)

            @pl.when(r == 1)
            def _():
                reduce_chunk(rows_b, out_b)
                issue_out(j, out_b)

            return carry

        lax.fori_loop(0, per_w, chunk_body, 0)

        drain_out()
        drain_out()

    return sc_kernel


_SC_KERNEL = _make_sc_kernel()


@jax.jit
def kernel(feat_table, neigh_idx):
    packed = lax.bitcast_convert_type(
        feat_table.astype(jnp.bfloat16)[:, _PERM].reshape(N_NODES_C, W_C, 2),
        jnp.int32)
    neigh_flat = neigh_idx.reshape(-1)
    return _SC_KERNEL(packed, neigh_flat)


# speedup vs baseline: 2.7723x; 2.7723x over previous
import functools

import jax
import jax.numpy as jnp
import numpy as np
from jax import lax
from jax.experimental import pallas as pl
from jax.experimental.pallas import tpu as pltpu
from jax.experimental.pallas import tpu_sc as plsc

N_NODES_C = 100000
B_C = 50000
K_C = 10
D_C = 128
W_C = D_C // 2

CHUNK_B = 64
CHUNK_I = CHUNK_B * K_C
GATHER_SLICE = 128
N_GATHER = CHUNK_I // GATHER_SLICE
LANES = 16
H_VECS = W_C // LANES


def _col_interleave_perm():
    perm = np.empty(D_C, dtype=np.int32)
    for g in range(D_C // 32):
        for i in range(16):
            perm[32 * g + 2 * i] = 32 * g + i
            perm[32 * g + 2 * i + 1] = 32 * g + 16 + i
    return perm


_PERM = _col_interleave_perm()


def _make_sc_kernel():
    info = plsc.get_sparse_core_info()
    nc, ns = info.num_cores, info.num_subcores
    nw = nc * ns
    n_chunks = -(-B_C // CHUNK_B)
    per_w = -(-n_chunks // nw)
    last_base = B_C - CHUNK_B

    mesh = plsc.VectorSubcoreMesh(core_axis_name="c", subcore_axis_name="s")

    @functools.partial(
        pl.kernel,
        mesh=mesh,
        compiler_params=pltpu.CompilerParams(use_tc_tiling_on_sc=False),
        out_type=jax.ShapeDtypeStruct((B_C, D_C), jnp.float32),
        scratch_types=[
            pltpu.VMEM((CHUNK_I,), jnp.int32),
            pltpu.VMEM((CHUNK_I,), jnp.int32),
            pltpu.VMEM((CHUNK_I, W_C), jnp.int32),
            pltpu.VMEM((CHUNK_I, W_C), jnp.int32),
            pltpu.VMEM((CHUNK_B, D_C), jnp.float32),
            pltpu.VMEM((CHUNK_B, D_C), jnp.float32),
            pltpu.SemaphoreType.DMA,
            pltpu.SemaphoreType.DMA,
            pltpu.SemaphoreType.DMA,
        ],
    )
    def sc_kernel(table_hbm, neigh_hbm, out_hbm, idx_a, idx_b,
                  rows_a, rows_b, out_a, out_b, isem, gsem, osem):
        wid = lax.axis_index("s") * nc + lax.axis_index("c")
        inv_k = jnp.float32(1.0 / K_C)
        himask = jnp.int32(-65536)

        def lo_f32(v):
            return lax.bitcast_convert_type(lax.shift_left(v, 16),
                                            jnp.float32)

        def hi_f32(v):
            return lax.bitcast_convert_type(lax.bitwise_and(v, himask),
                                            jnp.float32)

        def chunk_base(j):
            return jnp.minimum((wid * per_w + j) * CHUNK_B, last_base)

        def issue_idx(j, idx_ref):
            base = chunk_base(j)
            pltpu.async_copy(
                neigh_hbm.at[pl.ds(base * K_C, CHUNK_I)], idx_ref, isem)

        def drain_idx():
            pltpu.make_async_copy(
                neigh_hbm.at[pl.ds(0, CHUNK_I)], idx_a, isem).wait()

        def issue_gathers(idx_ref, rows_ref):
            for g in range(N_GATHER):
                sl = pl.ds(g * GATHER_SLICE, GATHER_SLICE)
                pltpu.async_copy(
                    table_hbm.at[idx_ref.at[sl]], rows_ref.at[sl], gsem)

        def drain_gathers():
            pltpu.make_async_copy(
                table_hbm.at[pl.ds(0, CHUNK_I)], rows_a, gsem).wait()

        def issue_out(j, out_ref):
            base = chunk_base(j)
            pltpu.async_copy(
                out_ref, out_hbm.at[pl.ds(base, CHUNK_B)], osem)

        def drain_out():
            pltpu.make_async_copy(
                out_a, out_hbm.at[pl.ds(0, CHUNK_B)], osem).wait()

        def reduce_chunk(rv, ov):
            def tree_ops(l, b, h):
                t = {}
                lo_sl = pl.ds(32 * h, LANES)
                hi_sl = pl.ds(32 * h + LANES, LANES)

                def mk(name, f):
                    return lambda: t.__setitem__(name, f())

                ops = []
                for half, cvt in (("e", lo_f32), ("o", hi_f32)):
                    ops += [
                        mk(half + "a0", lambda c=cvt: c(l[0]) + c(l[1])),
                        mk(half + "a1", lambda c=cvt: c(l[2]) + c(l[3])),
                        mk(half + "a2", lambda c=cvt: c(l[4]) + c(l[5])),
                        mk(half + "a3", lambda c=cvt: c(l[6]) + c(l[7])),
                        mk(half + "a4", lambda c=cvt: c(l[8]) + c(l[9])),
                        mk(half + "b0",
                           lambda p=half: t[p + "a0"] + t[p + "a1"]),
                        mk(half + "b1",
                           lambda p=half: t[p + "a2"] + t[p + "a3"]),
                        mk(half + "c0",
                           lambda p=half: t[p + "b0"] + t[p + "b1"]),
                        mk(half + "c1",
                           lambda p=half: t[p + "c0"] + t[p + "a4"]),
                    ]
                ops.append(lambda: ov.__setitem__(
                    (b, lo_sl), t["ec1"] * inv_k))
                ops.append(lambda: ov.__setitem__(
                    (b, hi_sl), t["oc1"] * inv_k))
                return ops

            def center_body(i, carry2):
                b = i
                r0 = b * K_C
                pending = []
                for h in range(H_VECS):
                    hsl = pl.ds(h * LANES, LANES)
                    loads = []
                    for k in range(K_C):
                        loads.append(rv[r0 + k, hsl])
                        for _ in range(2):
                            if pending:
                                pending.pop(0)()
                    pending = tree_ops(loads, b, h)
                for op in pending:
                    op()
                return carry2

            lax.fori_loop(0, CHUNK_B, center_body, 0)

        issue_idx(0, idx_a)
        drain_idx()
        issue_idx(1, idx_b)
        issue_gathers(idx_a, rows_a)

        def chunk_body(j, carry):
            r = lax.rem(j, 2)
            nr = 1 - r

            drain_gathers()

            @pl.when(j < per_w - 2)
            def _():
                @pl.when(r == 0)
                def _():
                    issue_idx(j + 2, idx_a)

                @pl.when(r == 1)
                def _():
                    issue_idx(j + 2, idx_b)

            @pl.when(j < per_w - 1)
            def _():
                drain_idx()

                @pl.when(nr == 0)
                def _():
                    issue_gathers(idx_a, rows_a)

                @pl.when(nr == 1)
                def _():
                    issue_gathers(idx_b, rows_b)

            @pl.when(j >= 2)
            def _():
                drain_out()

            @pl.when(r == 0)
            def _():
                reduce_chunk(rows_a, out_a)
                issue_out(j, out_a)

            @pl.when(r == 1)
            def _():
                reduce_chunk(rows_b, out_b)
                issue_out(j, out_b)

            return carry

        lax.fori_loop(0, per_w, chunk_body, 0)

        drain_out()
        drain_out()

    return sc_kernel


_SC_KERNEL = _make_sc_kernel()


@jax.jit
def kernel(feat_table, neigh_idx):
    x = feat_table.astype(jnp.bfloat16).reshape(N_NODES_C, D_C // 32, 32)
    lo = lax.bitcast_convert_type(x[:, :, :16], jnp.uint16).astype(jnp.uint32)
    hi = lax.bitcast_convert_type(x[:, :, 16:], jnp.uint16).astype(jnp.uint32)
    packed = lax.bitcast_convert_type(
        lo | (hi << 16), jnp.int32).reshape(N_NODES_C, W_C)
    neigh_flat = neigh_idx.reshape(-1)
    return _SC_KERNEL(packed, neigh_flat)


# speedup vs baseline: 3.6982x; 1.3340x over previous
import functools

import jax
import jax.numpy as jnp
from jax import lax
from jax.experimental import pallas as pl
from jax.experimental.pallas import tpu as pltpu
from jax.experimental.pallas import tpu_sc as plsc

N_NODES_C = 100000
B_C = 50000
K_C = 10
D_C = 128

CHUNK_B = 32
CHUNK_I = CHUNK_B * K_C
GATHER_SLICE = 80
N_GATHER = CHUNK_I // GATHER_SLICE
LANES = 16
D_VECS = D_C // LANES
UNROLL = 4


def _make_sc_kernel():
    info = plsc.get_sparse_core_info()
    nc, ns = info.num_cores, info.num_subcores
    nw = nc * ns
    n_chunks = -(-B_C // CHUNK_B)
    per_w = -(-n_chunks // nw)
    last_base = B_C - CHUNK_B

    mesh = plsc.VectorSubcoreMesh(core_axis_name="c", subcore_axis_name="s")

    @functools.partial(
        pl.kernel,
        mesh=mesh,
        compiler_params=pltpu.CompilerParams(
            skip_device_barrier=True,
            disable_bounds_checks=True,
        ),
        out_type=jax.ShapeDtypeStruct((B_C, D_C), jnp.float32),
        scratch_types=[
            pltpu.VMEM((CHUNK_I,), jnp.int32),
            pltpu.VMEM((CHUNK_I,), jnp.int32),
            pltpu.VMEM((CHUNK_I, D_C), jnp.float32),
            pltpu.VMEM((CHUNK_I, D_C), jnp.float32),
            pltpu.VMEM((CHUNK_B, D_C), jnp.float32),
            pltpu.VMEM((CHUNK_B, D_C), jnp.float32),
            pltpu.SemaphoreType.DMA,
            pltpu.SemaphoreType.DMA,
            pltpu.SemaphoreType.DMA,
        ],
    )
    def sc_kernel(table_hbm, neigh_hbm, out_hbm, idx_a, idx_b,
                  rows_a, rows_b, out_a, out_b, isem, gsem, osem):
        wid = lax.axis_index("s") * nc + lax.axis_index("c")
        inv_k = jnp.float32(1.0 / K_C)

        def chunk_base(j):
            return jnp.minimum((wid * per_w + j) * CHUNK_B, last_base)

        def issue_idx(j, idx_ref):
            base = chunk_base(j)
            pltpu.async_copy(
                neigh_hbm.at[pl.ds(base * K_C, CHUNK_I)], idx_ref, isem)

        def drain_idx():
            pltpu.make_async_copy(
                neigh_hbm.at[pl.ds(0, CHUNK_I)], idx_a, isem).wait()

        def issue_gathers(idx_ref, rows_ref):
            for g in range(N_GATHER):
                sl = pl.ds(g * GATHER_SLICE, GATHER_SLICE)
                pltpu.async_copy(
                    table_hbm.at[idx_ref.at[sl]], rows_ref.at[sl], gsem)

        def drain_gathers():
            pltpu.make_async_copy(
                table_hbm.at[pl.ds(0, CHUNK_I)], rows_a, gsem).wait()

        def issue_out(j, out_ref):
            base = chunk_base(j)
            pltpu.async_copy(
                out_ref, out_hbm.at[pl.ds(base, CHUNK_B)], osem)

        def drain_out():
            pltpu.make_async_copy(
                out_a, out_hbm.at[pl.ds(0, CHUNK_B)], osem).wait()

        def reduce_chunk(rv, ov):
            def tree_ops(l, b, dsl):
                t = {}

                def fin():
                    t["c1"] = t["c0"] + t["a4"]
                    ov[b, dsl] = t["c1"] * inv_k

                return [
                    lambda: t.__setitem__("a0", l[0] + l[1]),
                    lambda: t.__setitem__("a1", l[2] + l[3]),
                    lambda: t.__setitem__("a2", l[4] + l[5]),
                    lambda: t.__setitem__("a3", l[6] + l[7]),
                    lambda: t.__setitem__("a4", l[8] + l[9]),
                    lambda: t.__setitem__("b0", t["a0"] + t["a1"]),
                    lambda: t.__setitem__("b1", t["a2"] + t["a3"]),
                    lambda: None,
                    lambda: t.__setitem__("c0", t["b0"] + t["b1"]),
                    fin,
                ]

            def center_body(i, carry2):
                b0 = i * UNROLL
                groups = [(b0 + u, d)
                          for u in range(UNROLL) for d in range(D_VECS)]
                pending = []
                for b, d in groups:
                    r0 = b * K_C
                    dsl = pl.ds(d * LANES, LANES)
                    loads = []
                    for k in range(K_C):
                        loads.append(rv[r0 + k, dsl])
                        if pending:
                            pending.pop(0)()
                    pending = tree_ops(loads, b, dsl)
                for op in pending:
                    op()
                return carry2

            lax.fori_loop(0, CHUNK_B // UNROLL, center_body, 0)

        issue_idx(0, idx_a)
        drain_idx()
        issue_idx(1, idx_b)
        issue_gathers(idx_a, rows_a)

        def chunk_body(j, carry):
            r = lax.rem(j, 2)
            nr = 1 - r

            drain_gathers()

            @pl.when(j < per_w - 2)
            def _():
                @pl.when(r == 0)
                def _():
                    issue_idx(j + 2, idx_a)

                @pl.when(r == 1)
                def _():
                    issue_idx(j + 2, idx_b)

            @pl.when(j < per_w - 1)
            def _():
                drain_idx()

                @pl.when(nr == 0)
                def _():
                    issue_gathers(idx_a, rows_a)

                @pl.when(nr == 1)
                def _():
                    issue_gathers(idx_b, rows_b)

            @pl.when(j >= 2)
            def _():
                drain_out()

            @pl.when(r == 0)
            def _():
                reduce_chunk(rows_a, out_a)
                issue_out(j, out_a)

            @pl.when(r == 1)
            def _():
                reduce_chunk(rows_b, out_b)
                issue_out(j, out_b)

            return carry

        lax.fori_loop(0, per_w, chunk_body, 0)

        drain_out()
        drain_out()

    return sc_kernel


_SC_KERNEL = _make_sc_kernel()


@jax.jit
def kernel(feat_table, neigh_idx):
    neigh_flat = neigh_idx.reshape(-1)
    return _SC_KERNEL(feat_table, neigh_flat)


# speedup vs baseline: 3.7986x; 1.0271x over previous
import functools

import jax
import jax.numpy as jnp
from jax import lax
from jax.experimental import pallas as pl
from jax.experimental.pallas import tpu as pltpu
from jax.experimental.pallas import tpu_sc as plsc

N_NODES_C = 100000
B_C = 50000
K_C = 10
D_C = 128

CHUNK_B = 40
CHUNK_I = CHUNK_B * K_C
GATHER_SLICE = 80
N_GATHER = CHUNK_I // GATHER_SLICE
LANES = 16
D_VECS = D_C // LANES
UNROLL = 4


def _make_sc_kernel():
    info = plsc.get_sparse_core_info()
    nc, ns = info.num_cores, info.num_subcores
    nw = nc * ns
    n_chunks = -(-B_C // CHUNK_B)
    per_w = -(-n_chunks // nw)
    last_base = B_C - CHUNK_B

    mesh = plsc.VectorSubcoreMesh(core_axis_name="c", subcore_axis_name="s")

    @functools.partial(
        pl.kernel,
        mesh=mesh,
        out_type=jax.ShapeDtypeStruct((B_C, D_C), jnp.float32),
        scratch_types=[
            pltpu.VMEM((CHUNK_I,), jnp.int32),
            pltpu.VMEM((CHUNK_I,), jnp.int32),
            pltpu.VMEM((CHUNK_I, D_C), jnp.float32),
            pltpu.VMEM((CHUNK_I, D_C), jnp.float32),
            pltpu.VMEM((CHUNK_B, D_C), jnp.float32),
            pltpu.VMEM((CHUNK_B, D_C), jnp.float32),
            pltpu.SemaphoreType.DMA,
            pltpu.SemaphoreType.DMA,
            pltpu.SemaphoreType.DMA,
        ],
    )
    def sc_kernel(table_hbm, neigh_hbm, out_hbm, idx_a, idx_b,
                  rows_a, rows_b, out_a, out_b, isem, gsem, osem):
        wid = lax.axis_index("s") * nc + lax.axis_index("c")
        inv_k = jnp.float32(1.0 / K_C)

        def chunk_base(j):
            return jnp.minimum((wid * per_w + j) * CHUNK_B, last_base)

        def issue_idx(j, idx_ref):
            base = chunk_base(j)
            pltpu.async_copy(
                neigh_hbm.at[pl.ds(base * K_C, CHUNK_I)], idx_ref, isem)

        def drain_idx():
            pltpu.make_async_copy(
                neigh_hbm.at[pl.ds(0, CHUNK_I)], idx_a, isem).wait()

        def issue_gathers(idx_ref, rows_ref):
            for g in range(N_GATHER):
                sl = pl.ds(g * GATHER_SLICE, GATHER_SLICE)
                pltpu.async_copy(
                    table_hbm.at[idx_ref.at[sl]], rows_ref.at[sl], gsem)

        def drain_gathers():
            pltpu.make_async_copy(
                table_hbm.at[pl.ds(0, CHUNK_I)], rows_a, gsem).wait()

        def issue_out(j, out_ref):
            base = chunk_base(j)
            pltpu.async_copy(
                out_ref, out_hbm.at[pl.ds(base, CHUNK_B)], osem)

        def drain_out():
            pltpu.make_async_copy(
                out_a, out_hbm.at[pl.ds(0, CHUNK_B)], osem).wait()

        def reduce_chunk(rv, ov):
            def tree_ops(l, b, dsl):
                t = {}

                def fin():
                    t["c1"] = t["c0"] + t["a4"]
                    ov[b, dsl] = t["c1"] * inv_k

                return [
                    lambda: t.__setitem__("a0", l[0] + l[1]),
                    lambda: t.__setitem__("a1", l[2] + l[3]),
                    lambda: t.__setitem__("a2", l[4] + l[5]),
                    lambda: t.__setitem__("a3", l[6] + l[7]),
                    lambda: t.__setitem__("a4", l[8] + l[9]),
                    lambda: t.__setitem__("b0", t["a0"] + t["a1"]),
                    lambda: t.__setitem__("b1", t["a2"] + t["a3"]),
                    lambda: None,
                    lambda: t.__setitem__("c0", t["b0"] + t["b1"]),
                    fin,
                ]

            def center_body(i, carry2):
                b0 = i * UNROLL
                groups = [(b0 + u, d)
                          for u in range(UNROLL) for d in range(D_VECS)]
                pending = []
                for b, d in groups:
                    r0 = b * K_C
                    dsl = pl.ds(d * LANES, LANES)
                    loads = []
                    for k in range(K_C):
                        loads.append(rv[r0 + k, dsl])
                        if pending:
                            pending.pop(0)()
                    pending = tree_ops(loads, b, dsl)
                for op in pending:
                    op()
                return carry2

            lax.fori_loop(0, CHUNK_B // UNROLL, center_body, 0)

        issue_idx(0, idx_a)
        drain_idx()
        issue_idx(1, idx_b)
        issue_gathers(idx_a, rows_a)

        def chunk_body(j, carry):
            r = lax.rem(j, 2)
            nr = 1 - r

            drain_gathers()

            @pl.when(j < per_w - 2)
            def _():
                @pl.when(r == 0)
                def _():
                    issue_idx(j + 2, idx_a)

                @pl.when(r == 1)
                def _():
                    issue_idx(j + 2, idx_b)

            @pl.when(j < per_w - 1)
            def _():
                drain_idx()

                @pl.when(nr == 0)
                def _():
                    issue_gathers(idx_a, rows_a)

                @pl.when(nr == 1)
                def _():
                    issue_gathers(idx_b, rows_b)

            @pl.when(j >= 2)
            def _():
                drain_out()

            @pl.when(r == 0)
            def _():
                reduce_chunk(rows_a, out_a)
                issue_out(j, out_a)

            @pl.when(r == 1)
            def _():
                reduce_chunk(rows_b, out_b)
                issue_out(j, out_b)

            return carry

        lax.fori_loop(0, per_w, chunk_body, 0)

        drain_out()
        drain_out()

    return sc_kernel


_SC_KERNEL = _make_sc_kernel()


@jax.jit
def kernel(feat_table, neigh_idx):
    neigh_flat = neigh_idx.reshape(-1)
    return _SC_KERNEL(feat_table, neigh_flat)
